# Initial kernel scaffold; baseline (speedup 1.0000x reference)
import jax
import jax.numpy as jnp
from jax.experimental import pallas as pl


def kernel(x, latent_pi, W, b, noise):
    raise NotImplementedError("write your pallas kernel here")


# speedup vs baseline: 1.1271x; 1.1271x over previous
import jax
import jax.numpy as jnp
from jax.experimental import pallas as pl
from jax.experimental.pallas import tpu as pltpu

_BATCH = 16384
_GROUPS = 20
_REPEAT = 4
_OUT_D = _GROUPS * _REPEAT
_BLOCK = 2048


def _body(x_ref, o_ref):
    x = jnp.clip(x_ref[...], -1.0, 1.0)
    xp = jnp.pad(x, ((0, 0), (0, _OUT_D - _GROUPS)))
    idx = jax.lax.broadcasted_iota(jnp.int32, (x.shape[0], _OUT_D), 1) // _REPEAT
    o_ref[...] = jnp.take_along_axis(xp, idx, axis=1)


def kernel(x, latent_pi, W, b, noise):
    del latent_pi, W, b, noise
    return pl.pallas_call(
        _body,
        grid=(_BATCH // _BLOCK,),
        in_specs=[pl.BlockSpec((_BLOCK, _GROUPS), lambda i: (i, 0))],
        out_specs=pl.BlockSpec((_BLOCK, _OUT_D), lambda i: (i, 0)),
        out_shape=jax.ShapeDtypeStruct((_BATCH, _OUT_D), jnp.float32),
        compiler_params=pltpu.CompilerParams(
            dimension_semantics=("arbitrary",),
        ),
    )(x)
